# Initial kernel scaffold; baseline (speedup 1.0000x reference)
import jax
import jax.numpy as jnp
from jax.experimental import pallas as pl


def kernel(x, edge_index, batch, gn0_w, gn0_b, gn0_ms, gn1_w, gn1_b, gn1_ms, gn2_w, gn2_b, gn2_ms, gn3_w, gn3_b, gn3_ms, W1, b1, W2, b2, W3, b3, gW1, gb1, gW2, gb2, gW3, gb3, lW1, lb1, lW2, lb2, lW3, lb3):
    raise NotImplementedError("write your pallas kernel here")


# speedup vs baseline: 9.4365x; 9.4365x over previous
import functools

import jax
import jax.numpy as jnp
from jax import lax
from jax.experimental import pallas as pl
from jax.experimental.pallas import tpu as pltpu
from jax.experimental.pallas import tpu_sc as plsc

N = 100000
E = 1600000
G = 64
D_IN = 4
H = 128
EPS = 1e-5

RB = 2000
NBLK = N // RB
NTILE = 16
NSC = 2
NPAD = 112000
ROWS_PER_TILE = NPAD // NTILE
DZ = 1400
NPB = NPAD // RB

f32 = jnp.float32


CW = 8
EB = 2000
EBA = 1000


def _sc_mesh():
    return plsc.VectorSubcoreMesh(core_axis_name="c", subcore_axis_name="s")


def _sc_degree(dst, ones_rows, zeros_rows):
    nblocks = E // (NSC * NTILE) // EB

    @functools.partial(
        pl.kernel,
        out_type=jax.ShapeDtypeStruct((NSC * NPAD, CW), f32),
        mesh=_sc_mesh(),
        compiler_params=pltpu.CompilerParams(use_tc_tiling_on_sc=False),
        scratch_types=[
            pltpu.VMEM((EB,), jnp.int32),
            pltpu.VMEM((EB, CW), f32),
            pltpu.VMEM((DZ, CW), f32),
            pltpu.VMEM_SHARED((NPAD, CW), f32),
        ],
    )
    def k(dst_hbm, ones_hbm, zeros_hbm, out_hbm, idx_v, ones_v, zv, acc):
        c = lax.axis_index("c")
        s = lax.axis_index("s")
        pltpu.sync_copy(ones_hbm, ones_v)
        pltpu.sync_copy(zeros_hbm, zv)

        @pl.loop(0, ROWS_PER_TILE // DZ)
        def _(j):
            st = pl.multiple_of(s * ROWS_PER_TILE + j * DZ, 8)
            pltpu.sync_copy(zv, acc.at[pl.ds(st, DZ)])

        plsc.subcore_barrier()
        tile_base = (c * NTILE + s) * (nblocks * EB)

        @pl.loop(0, nblocks)
        def _(j):
            pltpu.sync_copy(dst_hbm.at[pl.ds(tile_base + j * EB, EB)], idx_v)
            pltpu.sync_copy(ones_v, acc.at[idx_v], add=True)

        plsc.subcore_barrier()

        @pl.loop(0, ROWS_PER_TILE // DZ)
        def _(j):
            r0 = pl.multiple_of(s * ROWS_PER_TILE + j * DZ, 8)
            ro = pl.multiple_of(c * NPAD + s * ROWS_PER_TILE + j * DZ, 8)
            pltpu.sync_copy(acc.at[pl.ds(r0, DZ)], out_hbm.at[pl.ds(ro, DZ)])

    return k(dst, ones_rows, zeros_rows)


def _sc_agg1(u8, eb3, zeros_rows):
    nb = E // (NSC * NTILE) // EBA

    @functools.partial(
        pl.kernel,
        out_type=jax.ShapeDtypeStruct((NSC * NPAD, CW), f32),
        mesh=_sc_mesh(),
        compiler_params=pltpu.CompilerParams(use_tc_tiling_on_sc=False),
        scratch_types=[
            pltpu.VMEM((2, EBA), jnp.int32),
            pltpu.VMEM((2, EBA), jnp.int32),
            pltpu.VMEM((EBA, CW), f32),
            pltpu.VMEM((EBA, CW), f32),
            pltpu.VMEM((DZ, CW), f32),
            pltpu.VMEM_SHARED((NPAD, CW), f32),
            pltpu.SemaphoreType.DMA,
            pltpu.SemaphoreType.DMA,
        ],
    )
    def k(u_hbm, e_hbm, zeros_hbm, out_hbm,
          buf0, buf1, rows0, rows1, zv, acc, sg0, sg1):
        c = lax.axis_index("c")
        s = lax.axis_index("s")
        bufs = (buf0, buf1)
        rows = (rows0, rows1)
        sgs = (sg0, sg1)
        pltpu.sync_copy(zeros_hbm, zv)

        @pl.loop(0, ROWS_PER_TILE // DZ)
        def _(j):
            st = pl.multiple_of(s * ROWS_PER_TILE + j * DZ, 8)
            pltpu.sync_copy(zv, acc.at[pl.ds(st, DZ)])

        plsc.subcore_barrier()
        blk0 = (c * NTILE + s) * nb

        def start(j, b):
            pltpu.sync_copy(e_hbm.at[blk0 + j], bufs[b])
            pltpu.async_copy(u_hbm.at[bufs[b].at[0]], rows[b], sgs[b])

        def finish(b):
            pltpu.make_async_copy(u_hbm.at[bufs[b].at[0]], rows[b],
                                  sgs[b]).wait()
            pltpu.sync_copy(rows[b], acc.at[bufs[b].at[1]], add=True)

        start(0, 0)

        @pl.loop(0, nb // 2)
        def _(jj):
            j = jj * 2
            pltpu.make_async_copy(u_hbm.at[buf0.at[0]], rows0, sg0).wait()

            @pl.when(j + 1 < nb)
            def _():
                start(j + 1, 1)

            pltpu.sync_copy(rows0, acc.at[buf0.at[1]], add=True)

            @pl.when(j + 1 < nb)
            def _():
                pltpu.make_async_copy(u_hbm.at[buf1.at[0]], rows1,
                                      sg1).wait()

                @pl.when(j + 2 < nb)
                def _():
                    start(j + 2, 0)

                pltpu.sync_copy(rows1, acc.at[buf1.at[1]], add=True)

        plsc.subcore_barrier()

        @pl.loop(0, ROWS_PER_TILE // DZ)
        def _(j):
            r0 = pl.multiple_of(s * ROWS_PER_TILE + j * DZ, 8)
            ro = pl.multiple_of(c * NPAD + s * ROWS_PER_TILE + j * DZ, 8)
            pltpu.sync_copy(acc.at[pl.ds(r0, DZ)], out_hbm.at[pl.ds(ro, DZ)])

    return k(u8, eb3, zeros_rows)


def _sc_agg16(u2d, eb3, zeros_rows):
    nb = E // NTILE // EBA
    FCH = 16

    @functools.partial(
        pl.kernel,
        out_type=jax.ShapeDtypeStruct((NPAD * FCH, CW), f32),
        mesh=_sc_mesh(),
        compiler_params=pltpu.CompilerParams(use_tc_tiling_on_sc=False),
        scratch_types=[
            pltpu.VMEM((2, EBA), jnp.int32),
            pltpu.VMEM((2, EBA), jnp.int32),
            pltpu.VMEM((EBA,), jnp.int32),
            pltpu.VMEM((EBA,), jnp.int32),
            pltpu.VMEM((EBA, CW), f32),
            pltpu.VMEM((EBA, CW), f32),
            pltpu.VMEM((DZ,), jnp.int32),
            pltpu.VMEM((DZ, CW), f32),
            pltpu.VMEM((DZ, CW), f32),
            pltpu.VMEM_SHARED((NPAD, CW), f32),
            pltpu.SemaphoreType.DMA,
            pltpu.SemaphoreType.DMA,
        ],
    )
    def k(u_hbm, e_hbm, zeros_hbm, out_hbm,
          buf0, buf1, ig0, ig1, rows0, rows1, idxd_v, rowsd_v, zv, acc,
          sg0, sg1):
        c = lax.axis_index("c")
        s = lax.axis_index("s")
        bufs = (buf0, buf1)
        igs = (ig0, ig1)
        rows = (rows0, rows1)
        sgs = (sg0, sg1)
        pltpu.sync_copy(zeros_hbm, zv)

        @pl.loop(0, FCH // NSC)
        def _(kk):
            ch = NSC * kk + c

            @pl.loop(0, ROWS_PER_TILE // DZ)
            def _(j):
                st = pl.multiple_of(s * ROWS_PER_TILE + j * DZ, 8)
                pltpu.sync_copy(zv, acc.at[pl.ds(st, DZ)])

            plsc.subcore_barrier()
            blk0 = s * nb

            def start(j, b):
                pltpu.sync_copy(e_hbm.at[blk0 + j], bufs[b])

                @pl.loop(0, EBA, step=16)
                def _(t):
                    sv = bufs[b][0, pl.ds(t, 16)]
                    igs[b][pl.ds(t, 16)] = sv * FCH + ch

                pltpu.async_copy(u_hbm.at[igs[b]], rows[b], sgs[b])

            start(0, 0)

            @pl.loop(0, nb // 2)
            def _(jj):
                j = jj * 2
                pltpu.make_async_copy(u_hbm.at[ig0], rows0, sg0).wait()

                @pl.when(j + 1 < nb)
                def _():
                    start(j + 1, 1)

                pltpu.sync_copy(rows0, acc.at[buf0.at[1]], add=True)

                @pl.when(j + 1 < nb)
                def _():
                    pltpu.make_async_copy(u_hbm.at[ig1], rows1, sg1).wait()

                    @pl.when(j + 2 < nb)
                    def _():
                        start(j + 2, 0)

                    pltpu.sync_copy(rows1, acc.at[buf1.at[1]], add=True)

            plsc.subcore_barrier()
            @pl.loop(0, ROWS_PER_TILE // DZ)
            def _(j):
                r0 = pl.multiple_of(s * ROWS_PER_TILE + j * DZ, 8)

                @pl.loop(0, DZ, step=16)
                def _(t):
                    node = r0 + t + lax.iota(jnp.int32, 16)
                    idxd_v[pl.ds(t, 16)] = node * FCH + ch

                pltpu.sync_copy(acc.at[pl.ds(r0, DZ)], rowsd_v)
                pltpu.sync_copy(rowsd_v, out_hbm.at[idxd_v])

    return k(u2d, eb3, zeros_rows)


def _onehot(batch_col):
    io = lax.broadcasted_iota(jnp.int32, (batch_col.shape[0], G), 1)
    ohb = io == batch_col
    return ohb, ohb.astype(f32)


def _dotT(a, b):
    return lax.dot_general(a, b, (((0,), (0,)), ((), ())),
                           preferred_element_type=f32)


def _dot(a, b):
    return jnp.dot(a, b, preferred_element_type=f32)


def _ab_epilogue(s12, cntb, d, w, b, ms):
    cntc = jnp.maximum(cntb[:, :d], 1.0)
    m = s12[:, :d] / cntc
    ex2 = s12[:, d:2 * d] / cntc
    var = ex2 - m * m * ms * (2.0 - ms)
    sinv = lax.rsqrt(var + EPS)
    return w * sinv, b - w * ms * m * sinv


def _stats_accum(i, oh, h, s12_ref, cacc_ref=None):
    @pl.when(i == 0)
    def _():
        s12_ref[...] = jnp.zeros_like(s12_ref)
        if cacc_ref is not None:
            cacc_ref[...] = jnp.zeros_like(cacc_ref)

    s12_ref[...] += _dotT(oh, jnp.concatenate([h, h * h], axis=1))
    if cacc_ref is not None:
        cacc_ref[...] += _dotT(oh, jnp.ones((oh.shape[0], 128), f32))


def _row_spec(d):
    return pl.BlockSpec((RB, d), lambda i: (i, 0))


def _full_spec(shape):
    nd = len(shape)
    return pl.BlockSpec(shape, lambda i: (0,) * nd)


def _tc_stats0(x, batch2, w, b, ms):
    def body(x_ref, bt_ref, w_ref, b_ref, ms_ref,
             A_ref, B_ref, cnt_ref, s12, cacc):
        i = pl.program_id(0)
        _, oh = _onehot(bt_ref[...])
        _stats_accum(i, oh, x_ref[...], s12, cacc)

        @pl.when(i == NBLK - 1)
        def _():
            cnt_ref[...] = cacc[...]
            A, B = _ab_epilogue(s12[...], cacc[...], D_IN,
                                w_ref[...], b_ref[...], ms_ref[...])
            A_ref[...] = A
            B_ref[...] = B

    return pl.pallas_call(
        body,
        grid=(NBLK,),
        in_specs=[_row_spec(D_IN), _row_spec(1),
                  _full_spec((1, D_IN)), _full_spec((1, D_IN)),
                  _full_spec((1, D_IN))],
        out_specs=[_full_spec((G, D_IN)), _full_spec((G, D_IN)),
                   _full_spec((G, 128))],
        out_shape=[jax.ShapeDtypeStruct((G, D_IN), f32),
                   jax.ShapeDtypeStruct((G, D_IN), f32),
                   jax.ShapeDtypeStruct((G, 128), f32)],
        scratch_shapes=[pltpu.VMEM((G, 2 * D_IN), f32),
                        pltpu.VMEM((G, 128), f32)],
    )(x, batch2, w, b, ms)


def _tc_dinv(degp):
    def body(p0_ref, p1_ref, dinv_ref):
        deg = 1.0 + p0_ref[:, 0:1] + p1_ref[:, 0:1]
        dinv_ref[...] = lax.rsqrt(deg)

    return pl.pallas_call(
        body,
        grid=(NBLK,),
        in_specs=[pl.BlockSpec((RB, 8), lambda i: (i, 0)),
                  pl.BlockSpec((RB, 8), lambda i: (i + NPB, 0))],
        out_specs=_row_spec(1),
        out_shape=jax.ShapeDtypeStruct((N, 1), f32),
    )(degp, degp)


def _tc_transform1(x, batch2, A0, B0, dinv):
    def body(x_ref, bt_ref, A_ref, B_ref, dv_ref, u_ref):
        _, oh = _onehot(bt_ref[...])
        z = _dot(oh, A_ref[...]) * x_ref[...] + _dot(oh, B_ref[...])
        zd = z * dv_ref[...]
        u_ref[...] = jnp.concatenate(
            [zd, jnp.zeros((RB, 8 - D_IN), f32)], axis=1)

    return pl.pallas_call(
        body,
        grid=(NBLK,),
        in_specs=[_row_spec(D_IN), _row_spec(1),
                  _full_spec((G, D_IN)), _full_spec((G, D_IN)),
                  _row_spec(1)],
        out_specs=_row_spec(8),
        out_shape=jax.ShapeDtypeStruct((N, 8), f32),
    )(x, batch2, A0, B0, dinv)


def _tc_epi1(aggp, u1, dinv, W1, b1, batch2, cnt, w, b, ms):
    def body(a0_ref, a1_ref, u_ref, dv_ref, W_ref, bv_ref, bt_ref,
             cnt_ref, w_ref, b_ref, ms_ref, h_ref, A_ref, B_ref, s12):
        i = pl.program_id(0)
        y = (a0_ref[...] + a1_ref[...] + u_ref[...]) * dv_ref[...]
        h = _dot(y[:, :D_IN], W_ref[...]) + bv_ref[...]
        h_ref[...] = h
        _, oh = _onehot(bt_ref[...])
        _stats_accum(i, oh, h, s12)

        @pl.when(i == NBLK - 1)
        def _():
            A, B = _ab_epilogue(s12[...], cnt_ref[...], H,
                                w_ref[...], b_ref[...], ms_ref[...])
            A_ref[...] = A
            B_ref[...] = B

    return pl.pallas_call(
        body,
        grid=(NBLK,),
        in_specs=[pl.BlockSpec((RB, 8), lambda i: (i, 0)),
                  pl.BlockSpec((RB, 8), lambda i: (i + NPB, 0)),
                  _row_spec(8), _row_spec(1),
                  _full_spec((D_IN, H)), _full_spec((1, H)),
                  _row_spec(1), _full_spec((G, 128)),
                  _full_spec((1, H)), _full_spec((1, H)),
                  _full_spec((1, H))],
        out_specs=[_row_spec(H), _full_spec((G, H)), _full_spec((G, H))],
        out_shape=[jax.ShapeDtypeStruct((N, H), f32),
                   jax.ShapeDtypeStruct((G, H), f32),
                   jax.ShapeDtypeStruct((G, H), f32)],
        scratch_shapes=[pltpu.VMEM((G, 2 * H), f32)],
    )(aggp, aggp, u1, dinv, W1, b1, batch2, cnt, w, b, ms)


def _tc_transform23(h, batch2, A, B, dinv, W):
    def body(h_ref, bt_ref, A_ref, B_ref, dv_ref, W_ref, u_ref):
        _, oh = _onehot(bt_ref[...])
        z = _dot(oh, A_ref[...]) * h_ref[...] + _dot(oh, B_ref[...])
        z = jnp.maximum(z, 0.0)
        u_ref[...] = _dot(z, W_ref[...]) * dv_ref[...]

    return pl.pallas_call(
        body,
        grid=(NBLK,),
        in_specs=[_row_spec(H), _row_spec(1),
                  _full_spec((G, H)), _full_spec((G, H)),
                  _row_spec(1), _full_spec((H, H))],
        out_specs=_row_spec(H),
        out_shape=jax.ShapeDtypeStruct((N, H), f32),
    )(h, batch2, A, B, dinv, W)


def _tc_epi23(agg, u, dinv, bvec, batch2, cnt, w, b, ms):
    def body(a_ref, u_ref, dv_ref, bv_ref, bt_ref, cnt_ref,
             w_ref, b_ref, ms_ref, h_ref, A_ref, B_ref, s12):
        i = pl.program_id(0)
        h = (a_ref[...] + u_ref[...]) * dv_ref[...] + bv_ref[...]
        h_ref[...] = h
        _, oh = _onehot(bt_ref[...])
        _stats_accum(i, oh, h, s12)

        @pl.when(i == NBLK - 1)
        def _():
            A, B = _ab_epilogue(s12[...], cnt_ref[...], H,
                                w_ref[...], b_ref[...], ms_ref[...])
            A_ref[...] = A
            B_ref[...] = B

    return pl.pallas_call(
        body,
        grid=(NBLK,),
        in_specs=[_row_spec(H), _row_spec(H), _row_spec(1),
                  _full_spec((1, H)), _row_spec(1), _full_spec((G, 128)),
                  _full_spec((1, H)), _full_spec((1, H)),
                  _full_spec((1, H))],
        out_specs=[_row_spec(H), _full_spec((G, H)), _full_spec((G, H))],
        out_shape=[jax.ShapeDtypeStruct((N, H), f32),
                   jax.ShapeDtypeStruct((G, H), f32),
                   jax.ShapeDtypeStruct((G, H), f32)],
        scratch_shapes=[pltpu.VMEM((G, 2 * H), f32)],
    )(agg, u, dinv, bvec, batch2, cnt, w, b, ms)


def _tc_gmlp(h3, batch2, A3, B3, gW1, gb1, gW2, gb2, gW3, gb3):
    def body(h_ref, bt_ref, A_ref, B_ref, w1_ref, b1_ref, w2_ref,
             b2_ref, w3_ref, b3_ref, hn_ref, g_ref, gmax_ref, mx):
        i = pl.program_id(0)
        ohb, oh = _onehot(bt_ref[...])
        hn = _dot(oh, A_ref[...]) * h_ref[...] + _dot(oh, B_ref[...])
        hn_ref[...] = hn
        g1 = jnp.maximum(_dot(hn, w1_ref[...]) + b1_ref[...], 0.0)
        g2 = jnp.maximum(_dot(g1, w2_ref[...]) + b2_ref[...], 0.0)
        gg = _dot(g2, w3_ref[...]) + b3_ref[...]
        g_ref[...] = gg

        @pl.when(i == 0)
        def _():
            mx[...] = jnp.full_like(mx, -jnp.inf)

        vals = jnp.where(ohb, gg, -jnp.inf)
        pm = jnp.max(vals, axis=0, keepdims=True)
        mx[...] = jnp.maximum(mx[...], jnp.broadcast_to(pm, mx.shape))

        @pl.when(i == NBLK - 1)
        def _():
            gmax_ref[...] = jnp.where(jnp.isfinite(mx[...]), mx[...], 0.0)

    return pl.pallas_call(
        body,
        grid=(NBLK,),
        in_specs=[_row_spec(H), _row_spec(1),
                  _full_spec((G, H)), _full_spec((G, H)),
                  _full_spec((H, H)), _full_spec((1, H)),
                  _full_spec((H, H)), _full_spec((1, H)),
                  _full_spec((H, 1)), _full_spec((1, 1))],
        out_specs=[_row_spec(H), _row_spec(1), _full_spec((8, G))],
        out_shape=[jax.ShapeDtypeStruct((N, H), f32),
                   jax.ShapeDtypeStruct((N, 1), f32),
                   jax.ShapeDtypeStruct((8, G), f32)],
        scratch_shapes=[pltpu.VMEM((8, G), f32)],
    )(h3, batch2, A3, B3, gW1, gb1, gW2, gb2, gW3, gb3)


def _tc_attn(h3n, g, gmax, batch2):
    def body(h_ref, g_ref, gm_ref, bt_ref, out_ref, num, den):
        i = pl.program_id(0)
        ohb, oh = _onehot(bt_ref[...])

        @pl.when(i == 0)
        def _():
            num[...] = jnp.zeros_like(num)
            den[...] = jnp.zeros_like(den)

        gm_rows = jnp.max(jnp.where(ohb, gm_ref[0:1, :], -jnp.inf),
                          axis=1, keepdims=True)
        e = jnp.exp(g_ref[...] - gm_rows)
        num[...] += _dotT(oh, e * h_ref[...])
        den[...] += _dotT(oh, jnp.broadcast_to(e, (RB, H)))

        @pl.when(i == NBLK - 1)
        def _():
            out_ref[...] = num[...] / (den[...] + 1e-16)

    return pl.pallas_call(
        body,
        grid=(NBLK,),
        in_specs=[_row_spec(H), _row_spec(1), _full_spec((8, G)),
                  _row_spec(1)],
        out_specs=_full_spec((G, H)),
        out_shape=jax.ShapeDtypeStruct((G, H), f32),
        scratch_shapes=[pltpu.VMEM((G, H), f32), pltpu.VMEM((G, H), f32)],
    )(h3n, g, gmax, batch2)


def _tc_head(pooled, lW1, lb1, lW2, lb2, lW3, lb3):
    def body(p_ref, w1_ref, b1_ref, w2_ref, b2_ref, w3_ref, b3_ref,
             out_ref):
        r = jnp.maximum(_dot(p_ref[...], w1_ref[...]) + b1_ref[...], 0.0)
        r = jnp.maximum(_dot(r, w2_ref[...]) + b2_ref[...], 0.0)
        out_ref[...] = _dot(r, w3_ref[...]) + b3_ref[...]

    return pl.pallas_call(
        body,
        grid=(1,),
        in_specs=[_full_spec((G, H)), _full_spec((H, H)),
                  _full_spec((1, H)), _full_spec((H, H)),
                  _full_spec((1, H)), _full_spec((H, 1)),
                  _full_spec((1, 1))],
        out_specs=_full_spec((G, 1)),
        out_shape=jax.ShapeDtypeStruct((G, 1), f32),
    )(pooled, lW1, lb1, lW2, lb2, lW3, lb3)


def kernel(x, edge_index, batch, gn0_w, gn0_b, gn0_ms, gn1_w, gn1_b,
           gn1_ms, gn2_w, gn2_b, gn2_ms, gn3_w, gn3_b, gn3_ms, W1, b1,
           W2, b2, W3, b3, gW1, gb1, gW2, gb2, gW3, gb3, lW1, lb1, lW2,
           lb2, lW3, lb3):
    src = edge_index[0]
    dst = edge_index[1]
    eb3 = jnp.stack([src.reshape(E // EBA, EBA),
                     dst.reshape(E // EBA, EBA)], axis=1)
    batch2 = batch.reshape(N, 1)
    r1 = lambda v: v.reshape(1, -1)
    zeros_rows = jnp.zeros((DZ, 8), f32)
    ones_rows = jnp.ones((2000, 8), f32)

    degp = _sc_degree(dst, ones_rows, zeros_rows)
    dinv = _tc_dinv(degp)

    A0, B0, cnt = _tc_stats0(x, batch2, r1(gn0_w), r1(gn0_b), r1(gn0_ms))
    u1 = _tc_transform1(x, batch2, A0, B0, dinv)
    agg1p = _sc_agg1(u1, eb3, zeros_rows)
    h1, A1, B1 = _tc_epi1(agg1p, u1, dinv, W1, r1(b1), batch2, cnt,
                          r1(gn1_w), r1(gn1_b), r1(gn1_ms))

    u2 = _tc_transform23(h1, batch2, A1, B1, dinv, W2)
    agg2 = _sc_agg16(u2.reshape(N * 16, 8), eb3, zeros_rows)
    h2, A2, B2 = _tc_epi23(agg2.reshape(NPAD, H), u2, dinv, r1(b2), batch2,
                           cnt, r1(gn2_w), r1(gn2_b), r1(gn2_ms))

    u3 = _tc_transform23(h2, batch2, A2, B2, dinv, W3)
    agg3 = _sc_agg16(u3.reshape(N * 16, 8), eb3, zeros_rows)
    h3, A3, B3 = _tc_epi23(agg3.reshape(NPAD, H), u3, dinv, r1(b3), batch2,
                           cnt, r1(gn3_w), r1(gn3_b), r1(gn3_ms))

    h3n, g, gmax = _tc_gmlp(h3, batch2, A3, B3, gW1, r1(gb1), gW2,
                            r1(gb2), gW3, r1(gb3))
    pooled = _tc_attn(h3n, g, gmax, batch2)
    return _tc_head(pooled, lW1, r1(lb1), lW2, r1(lb2), lW3,
                    lb3.reshape(1, 1))
